# baseline (device time: 855286 ns/iter reference)
import jax
import jax.numpy as jnp
from jax import lax
from jax.experimental import pallas as pl
from jax.experimental.pallas import tpu as pltpu

N_DEV = 8
S = 4096
D = 1024
BLK = 512
NB = S // BLK
H_LOC = 8
DH = 128
SCALE = 0.08838834764831843
EPS = 1e-5
BF = jnp.bfloat16
F32 = jnp.float32


def _ln_mod(xb, s_ref, sh_ref):
    mu = jnp.mean(xb, axis=-1, keepdims=True)
    var = jnp.mean((xb - mu) ** 2, axis=-1, keepdims=True)
    xn = (xb - mu) * lax.rsqrt(var + EPS)
    return xn * (1.0 + s_ref[...]) + sh_ref[...]


def _qkv_body(x_ref, sa_ref, sha_ref, wq_ref, wk_ref, wv_ref,
              q_ref, k_ref, v_ref):
    xm = _ln_mod(x_ref[...], sa_ref, sha_ref).astype(BF)
    q_ref[...] = jnp.dot(xm, wq_ref[...], preferred_element_type=F32).astype(BF)
    k_ref[...] = jnp.dot(xm, wk_ref[...], preferred_element_type=F32).astype(BF)
    v_ref[...] = jnp.dot(xm, wv_ref[...], preferred_element_type=F32).astype(BF)


def _attn_body(q_ref, k_ref, v_ref, o_ref):
    q = q_ref[...]
    k = k_ref[...]
    s = lax.dot_general(q, k, (((1,), (1,)), ((), ())),
                        preferred_element_type=F32) * SCALE
    m = jnp.max(s, axis=-1, keepdims=True)
    p = jnp.exp(s - m)
    l = jnp.sum(p, axis=-1, keepdims=True)
    o = jnp.dot(p.astype(BF), v_ref[...], preferred_element_type=F32)
    o_ref[...] = (o / l).astype(BF)


def _proj_body(a_ref, w_ref, o_ref):
    o_ref[...] = jnp.dot(a_ref[...], w_ref[...],
                         preferred_element_type=F32).astype(BF)


def _ffn_body(x1_ref, sm_ref, shm_ref, w1_ref, w2_ref, o_ref):
    xb = x1_ref[...].astype(F32)
    xm = _ln_mod(xb, sm_ref, shm_ref).astype(BF)
    h = jnp.dot(xm, w1_ref[...], preferred_element_type=F32)
    h = h * jax.nn.sigmoid(h)
    o_ref[...] = jnp.dot(h.astype(BF), w2_ref[...],
                         preferred_element_type=F32).astype(BF)


def _ar_body(p_ref, res_ref, g_ref, out_ref,
             comm_ref, rs_send, rs_recv, ag_send, ag_recv):
    d = lax.axis_index("i")
    left = lax.rem(d - 1 + N_DEV, N_DEV)
    right = lax.rem(d + 1, N_DEV)

    barrier = pltpu.get_barrier_semaphore()
    for nbr in (left, right):
        pl.semaphore_signal(barrier, inc=1, device_id=(nbr,),
                            device_id_type=pl.DeviceIdType.MESH)
    pl.semaphore_wait(barrier, 2)

    def chunk(i):
        return pl.ds(i * BLK, BLK)

    rdma = pltpu.make_async_remote_copy(
        src_ref=p_ref.at[chunk(d)],
        dst_ref=comm_ref.at[0],
        send_sem=rs_send.at[0],
        recv_sem=rs_recv.at[0],
        device_id=(right,),
        device_id_type=pl.DeviceIdType.MESH,
    )
    rdma.start()
    rdma.wait()
    for s in range(1, N_DEV - 1):
        cidx = lax.rem(d - s + N_DEV, N_DEV)
        comm_ref[s - 1] = comm_ref[s - 1] + p_ref[chunk(cidx), :]
        rdma = pltpu.make_async_remote_copy(
            src_ref=comm_ref.at[s - 1],
            dst_ref=comm_ref.at[s],
            send_sem=rs_send.at[s],
            recv_sem=rs_recv.at[s],
            device_id=(right,),
            device_id_type=pl.DeviceIdType.MESH,
        )
        rdma.start()
        rdma.wait()

    own = lax.rem(d + 1, N_DEV)
    total = comm_ref[N_DEV - 2].astype(F32) + p_ref[chunk(own), :].astype(F32)
    final = res_ref[chunk(own), :].astype(F32) + g_ref[...] * total
    out_ref[chunk(own), :] = final.astype(BF)

    for t in range(N_DEV - 1):
        ca = lax.rem(d + 1 - t + N_DEV, N_DEV)
        rdma = pltpu.make_async_remote_copy(
            src_ref=out_ref.at[chunk(ca)],
            dst_ref=out_ref.at[chunk(ca)],
            send_sem=ag_send.at[t],
            recv_sem=ag_recv.at[t],
            device_id=(right,),
            device_id_type=pl.DeviceIdType.MESH,
        )
        rdma.start()
        rdma.wait()

    def _second(second_barrier):
        for nbr in (left, right):
            pl.semaphore_signal(second_barrier, inc=1, device_id=(nbr,),
                                device_id_type=pl.DeviceIdType.MESH)
        pl.semaphore_wait(second_barrier, 2)

    pl.run_scoped(_second, second_barrier=pltpu.SemaphoreType.REGULAR)


def _all_reduce(partial, res, gate, collective_id):
    return pl.pallas_call(
        _ar_body,
        out_shape=jax.ShapeDtypeStruct((S, D), BF),
        in_specs=[pl.BlockSpec(memory_space=pltpu.VMEM)] * 3,
        out_specs=pl.BlockSpec(memory_space=pltpu.VMEM),
        scratch_shapes=[
            pltpu.VMEM((N_DEV - 1, BLK, D), BF),
            pltpu.SemaphoreType.DMA((N_DEV - 1,)),
            pltpu.SemaphoreType.DMA((N_DEV - 1,)),
            pltpu.SemaphoreType.DMA((N_DEV - 1,)),
            pltpu.SemaphoreType.DMA((N_DEV - 1,)),
        ],
        compiler_params=pltpu.CompilerParams(collective_id=collective_id),
    )(partial, res, gate)


def kernel(x, Wq, Wk, Wv, Wo, t_emb, W_mod, W_ff1, W_ff2):
    x0 = x[0]
    mod = t_emb @ W_mod
    sa, sha, ga, sm, shm, gm = jnp.split(mod, 6, axis=-1)

    Wq_b = Wq.astype(BF)
    Wk_b = Wk.astype(BF)
    Wv_b = Wv.astype(BF)
    Wo_b = Wo.astype(BF)
    W1_b = W_ff1.astype(BF)
    W2_b = W_ff2.astype(BF)

    row = pl.BlockSpec((BLK, D), lambda i: (i, 0))
    vec = pl.BlockSpec((1, D), lambda i: (0, 0))
    wfull = pl.BlockSpec((D, D), lambda i: (0, 0))

    q, k, v = pl.pallas_call(
        _qkv_body,
        grid=(NB,),
        in_specs=[row, vec, vec, wfull, wfull, wfull],
        out_specs=[row, row, row],
        out_shape=[jax.ShapeDtypeStruct((S, D), BF)] * 3,
    )(x0, sa, sha, Wq_b, Wk_b, Wv_b)

    attn = pl.pallas_call(
        _attn_body,
        grid=(H_LOC, NB),
        in_specs=[
            pl.BlockSpec((BLK, DH), lambda h, i: (i, h)),
            pl.BlockSpec((S, DH), lambda h, i: (0, h)),
            pl.BlockSpec((S, DH), lambda h, i: (0, h)),
        ],
        out_specs=pl.BlockSpec((BLK, DH), lambda h, i: (i, h)),
        out_shape=jax.ShapeDtypeStruct((S, D), BF),
    )(q, k, v)

    partial1 = pl.pallas_call(
        _proj_body,
        grid=(NB,),
        in_specs=[row, wfull],
        out_specs=row,
        out_shape=jax.ShapeDtypeStruct((S, D), BF),
    )(attn, Wo_b)

    x1 = _all_reduce(partial1, x0.astype(BF), ga, collective_id=0)

    partial2 = pl.pallas_call(
        _ffn_body,
        grid=(NB,),
        in_specs=[
            row, vec, vec,
            pl.BlockSpec((D, 4 * D // N_DEV), lambda i: (0, 0)),
            pl.BlockSpec((4 * D // N_DEV, D), lambda i: (0, 0)),
        ],
        out_specs=row,
        out_shape=jax.ShapeDtypeStruct((S, D), BF),
    )(x1, sm, shm, W1_b, W2_b)

    out = _all_reduce(partial2, x1, gm, collective_id=1)
    return out.astype(F32)[None]


# device time: 485310 ns/iter; 1.7623x vs baseline; 1.7623x over previous
import jax
import jax.numpy as jnp
from jax import lax
from jax.experimental import pallas as pl
from jax.experimental.pallas import tpu as pltpu

N_DEV = 8
S = 4096
D = 1024
BLK = 512
NB = S // BLK
H_LOC = 8
DH = 128
SCALE = 0.08838834764831843
EPS = 1e-5
BF = jnp.bfloat16
F32 = jnp.float32


def _ln_mod(xb, s_ref, sh_ref):
    mu = jnp.mean(xb, axis=-1, keepdims=True)
    var = jnp.mean((xb - mu) ** 2, axis=-1, keepdims=True)
    xn = (xb - mu) * lax.rsqrt(var + EPS)
    return xn * (1.0 + s_ref[...]) + sh_ref[...]


def _qkv_body(x_ref, sa_ref, sha_ref, wq_ref, wk_ref, wv_ref,
              q_ref, k_ref, v_ref):
    xm = _ln_mod(x_ref[...], sa_ref, sha_ref).astype(BF)
    q_ref[...] = jnp.dot(xm, wq_ref[...], preferred_element_type=F32).astype(BF)
    k_ref[...] = jnp.dot(xm, wk_ref[...], preferred_element_type=F32).astype(BF)
    v_ref[...] = jnp.dot(xm, wv_ref[...], preferred_element_type=F32).astype(BF)


def _attn_body(q_ref, k_ref, v_ref, o_ref):
    q = q_ref[...]
    k = k_ref[...]
    s = lax.dot_general(q, k, (((1,), (1,)), ((), ())),
                        preferred_element_type=F32)
    p = jnp.exp((s * SCALE).astype(BF))
    l = jnp.sum(p, axis=-1, keepdims=True, dtype=F32)
    o = jnp.dot(p, v_ref[...], preferred_element_type=F32)
    o_ref[...] = (o / l).astype(BF)


def _proj_body(a_ref, w_ref, o_ref):
    o_ref[...] = jnp.dot(a_ref[...], w_ref[...],
                         preferred_element_type=F32).astype(BF)


def _ffn_body(x1_ref, sm_ref, shm_ref, w1_ref, w2_ref, o_ref):
    xb = x1_ref[...].astype(F32)
    xm = _ln_mod(xb, sm_ref, shm_ref).astype(BF)
    h = jnp.dot(xm, w1_ref[...], preferred_element_type=F32)
    h = h * jax.nn.sigmoid(h)
    o_ref[...] = jnp.dot(h.astype(BF), w2_ref[...],
                         preferred_element_type=F32).astype(BF)


HALF = D // 2


def _ar_body(p_ref, res_ref, g_ref, out_ref,
             comm_r, comm_l, rs_send_r, rs_recv_r, rs_send_l, rs_recv_l,
             ag_send_r, ag_recv_r, ag_send_l, ag_recv_l):
    d = lax.axis_index("i")
    left = lax.rem(d - 1 + N_DEV, N_DEV)
    right = lax.rem(d + 1, N_DEV)

    barrier = pltpu.get_barrier_semaphore()
    for nbr in (left, right):
        pl.semaphore_signal(barrier, inc=1, device_id=(nbr,),
                            device_id_type=pl.DeviceIdType.MESH)
    pl.semaphore_wait(barrier, 2)

    def rows(i):
        return pl.ds(i * BLK, BLK)

    ca_r = pl.ds(0, HALF)
    ca_l = pl.ds(HALF, HALF)

    for s in range(N_DEV - 1):
        cr = lax.rem(d - s + N_DEV, N_DEV)
        cl = lax.rem(d + s, N_DEV)
        if s == 0:
            src_r = p_ref.at[rows(cr), ca_r]
            src_l = p_ref.at[rows(cl), ca_l]
        else:
            comm_r[s - 1] = comm_r[s - 1] + p_ref[rows(cr), ca_r]
            comm_l[s - 1] = comm_l[s - 1] + p_ref[rows(cl), ca_l]
            src_r = comm_r.at[s - 1]
            src_l = comm_l.at[s - 1]
        rdma_r = pltpu.make_async_remote_copy(
            src_ref=src_r, dst_ref=comm_r.at[s],
            send_sem=rs_send_r.at[s], recv_sem=rs_recv_r.at[s],
            device_id=(right,), device_id_type=pl.DeviceIdType.MESH,
        )
        rdma_l = pltpu.make_async_remote_copy(
            src_ref=src_l, dst_ref=comm_l.at[s],
            send_sem=rs_send_l.at[s], recv_sem=rs_recv_l.at[s],
            device_id=(left,), device_id_type=pl.DeviceIdType.MESH,
        )
        rdma_r.start()
        rdma_l.start()
        rdma_r.wait()
        rdma_l.wait()

    own_r = lax.rem(d + 1, N_DEV)
    own_l = lax.rem(d - 1 + N_DEV, N_DEV)
    tot_r = comm_r[N_DEV - 2].astype(F32) + p_ref[rows(own_r), ca_r].astype(F32)
    tot_l = comm_l[N_DEV - 2].astype(F32) + p_ref[rows(own_l), ca_l].astype(F32)
    g = g_ref[...]
    out_ref[rows(own_r), ca_r] = (
        res_ref[rows(own_r), ca_r].astype(F32) + g[:, :HALF] * tot_r
    ).astype(BF)
    out_ref[rows(own_l), ca_l] = (
        res_ref[rows(own_l), ca_l].astype(F32) + g[:, HALF:] * tot_l
    ).astype(BF)

    for t in range(N_DEV - 1):
        cr = lax.rem(d + 1 - t + N_DEV, N_DEV)
        cl = lax.rem(d - 1 + t + N_DEV, N_DEV)
        rdma_r = pltpu.make_async_remote_copy(
            src_ref=out_ref.at[rows(cr), ca_r],
            dst_ref=out_ref.at[rows(cr), ca_r],
            send_sem=ag_send_r.at[t], recv_sem=ag_recv_r.at[t],
            device_id=(right,), device_id_type=pl.DeviceIdType.MESH,
        )
        rdma_l = pltpu.make_async_remote_copy(
            src_ref=out_ref.at[rows(cl), ca_l],
            dst_ref=out_ref.at[rows(cl), ca_l],
            send_sem=ag_send_l.at[t], recv_sem=ag_recv_l.at[t],
            device_id=(left,), device_id_type=pl.DeviceIdType.MESH,
        )
        rdma_r.start()
        rdma_l.start()
        rdma_r.wait()
        rdma_l.wait()

    def _second(second_barrier):
        for nbr in (left, right):
            pl.semaphore_signal(second_barrier, inc=1, device_id=(nbr,),
                                device_id_type=pl.DeviceIdType.MESH)
        pl.semaphore_wait(second_barrier, 2)

    pl.run_scoped(_second, second_barrier=pltpu.SemaphoreType.REGULAR)


def _all_reduce(partial, res, gate, collective_id):
    return pl.pallas_call(
        _ar_body,
        out_shape=jax.ShapeDtypeStruct((S, D), BF),
        in_specs=[pl.BlockSpec(memory_space=pltpu.VMEM)] * 3,
        out_specs=pl.BlockSpec(memory_space=pltpu.VMEM),
        scratch_shapes=[
            pltpu.VMEM((N_DEV - 1, BLK, HALF), BF),
            pltpu.VMEM((N_DEV - 1, BLK, HALF), BF),
        ] + [pltpu.SemaphoreType.DMA((N_DEV - 1,))] * 8,
        compiler_params=pltpu.CompilerParams(collective_id=collective_id),
    )(partial, res, gate)


def kernel(x, Wq, Wk, Wv, Wo, t_emb, W_mod, W_ff1, W_ff2):
    x0 = x[0]
    mod = t_emb @ W_mod
    sa, sha, ga, sm, shm, gm = jnp.split(mod, 6, axis=-1)

    Wq_b = Wq.astype(BF)
    Wk_b = Wk.astype(BF)
    Wv_b = Wv.astype(BF)
    Wo_b = Wo.astype(BF)
    W1_b = W_ff1.astype(BF)
    W2_b = W_ff2.astype(BF)

    row = pl.BlockSpec((BLK, D), lambda i: (i, 0))
    vec = pl.BlockSpec((1, D), lambda i: (0, 0))
    wfull = pl.BlockSpec((D, D), lambda i: (0, 0))

    q, k, v = pl.pallas_call(
        _qkv_body,
        grid=(NB,),
        in_specs=[row, vec, vec, wfull, wfull, wfull],
        out_specs=[row, row, row],
        out_shape=[jax.ShapeDtypeStruct((S, D), BF)] * 3,
    )(x0, sa, sha, Wq_b, Wk_b, Wv_b)

    attn = pl.pallas_call(
        _attn_body,
        grid=(H_LOC, NB),
        in_specs=[
            pl.BlockSpec((BLK, DH), lambda h, i: (i, h)),
            pl.BlockSpec((S, DH), lambda h, i: (0, h)),
            pl.BlockSpec((S, DH), lambda h, i: (0, h)),
        ],
        out_specs=pl.BlockSpec((BLK, DH), lambda h, i: (i, h)),
        out_shape=jax.ShapeDtypeStruct((S, D), BF),
    )(q, k, v)

    partial1 = pl.pallas_call(
        _proj_body,
        grid=(NB,),
        in_specs=[row, wfull],
        out_specs=row,
        out_shape=jax.ShapeDtypeStruct((S, D), BF),
    )(attn, Wo_b)

    x1 = _all_reduce(partial1, x0.astype(BF), ga, collective_id=0)

    partial2 = pl.pallas_call(
        _ffn_body,
        grid=(NB,),
        in_specs=[
            row, vec, vec,
            pl.BlockSpec((D, 4 * D // N_DEV), lambda i: (0, 0)),
            pl.BlockSpec((4 * D // N_DEV, D), lambda i: (0, 0)),
        ],
        out_specs=row,
        out_shape=jax.ShapeDtypeStruct((S, D), BF),
    )(x1, sm, shm, W1_b, W2_b)

    out = _all_reduce(partial2, x1, gm, collective_id=1)
    return out.astype(F32)[None]


# device time: 452492 ns/iter; 1.8902x vs baseline; 1.0725x over previous
import jax
import jax.numpy as jnp
from jax import lax
from jax.experimental import pallas as pl
from jax.experimental.pallas import tpu as pltpu

N_DEV = 8
S = 4096
D = 1024
BLK = 512
NB = S // BLK
H_LOC = 8
DH = 128
SCALE = 0.08838834764831843
EPS = 1e-5
BF = jnp.bfloat16
F32 = jnp.float32


def _ln_mod(xb, s_ref, sh_ref):
    mu = jnp.mean(xb, axis=-1, keepdims=True)
    var = jnp.mean((xb - mu) ** 2, axis=-1, keepdims=True)
    xn = (xb - mu) * lax.rsqrt(var + EPS)
    return xn * (1.0 + s_ref[...]) + sh_ref[...]


def _qkv_body(x_ref, sa_ref, sha_ref, wq_ref, wk_ref, wv_ref,
              q_ref, k_ref, v_ref):
    xm = _ln_mod(x_ref[...], sa_ref, sha_ref).astype(BF)
    q_ref[...] = (jnp.dot(xm, wq_ref[...], preferred_element_type=F32)
                  * SCALE).astype(BF)
    k_ref[...] = jnp.dot(xm, wk_ref[...], preferred_element_type=F32).astype(BF)
    v_ref[...] = jnp.dot(xm, wv_ref[...], preferred_element_type=F32).astype(BF)


def _attn_body(q_ref, k_ref, v_ref, o_ref):
    q = q_ref[...]
    k = k_ref[...]
    s = lax.dot_general(q, k, (((1,), (1,)), ((), ())),
                        preferred_element_type=F32)
    p = jnp.exp(s.astype(BF))
    l = jnp.sum(p, axis=-1, keepdims=True, dtype=F32)
    o = jnp.dot(p, v_ref[...], preferred_element_type=F32)
    o_ref[...] = (o / l).astype(BF)


def _proj_body(a_ref, w_ref, o_ref):
    o_ref[...] = jnp.dot(a_ref[...], w_ref[...],
                         preferred_element_type=F32).astype(BF)


def _ffn_body(x1_ref, sm_ref, shm_ref, w1_ref, w2_ref, o_ref):
    xb = x1_ref[...].astype(F32)
    xm = _ln_mod(xb, sm_ref, shm_ref).astype(BF)
    h = jnp.dot(xm, w1_ref[...], preferred_element_type=F32)
    h = h * jax.nn.sigmoid(h)
    o_ref[...] = jnp.dot(h.astype(BF), w2_ref[...],
                         preferred_element_type=F32).astype(BF)


HALF = D // 2


def _ar_body(p_ref, res_ref, g_ref, out_ref,
             comm_r, comm_l, rs_send_r, rs_recv_r, rs_send_l, rs_recv_l,
             ag_send_r, ag_recv_r, ag_send_l, ag_recv_l):
    d = lax.axis_index("i")
    left = lax.rem(d - 1 + N_DEV, N_DEV)
    right = lax.rem(d + 1, N_DEV)

    barrier = pltpu.get_barrier_semaphore()
    for nbr in (left, right):
        pl.semaphore_signal(barrier, inc=1, device_id=(nbr,),
                            device_id_type=pl.DeviceIdType.MESH)
    pl.semaphore_wait(barrier, 2)

    def rows(i):
        return pl.ds(i * BLK, BLK)

    ca_r = pl.ds(0, HALF)
    ca_l = pl.ds(HALF, HALF)

    for s in range(N_DEV - 1):
        cr = lax.rem(d - s + N_DEV, N_DEV)
        cl = lax.rem(d + s, N_DEV)
        if s == 0:
            src_r = p_ref.at[rows(cr), ca_r]
            src_l = p_ref.at[rows(cl), ca_l]
        else:
            comm_r[s - 1] = comm_r[s - 1] + p_ref[rows(cr), ca_r]
            comm_l[s - 1] = comm_l[s - 1] + p_ref[rows(cl), ca_l]
            src_r = comm_r.at[s - 1]
            src_l = comm_l.at[s - 1]
        rdma_r = pltpu.make_async_remote_copy(
            src_ref=src_r, dst_ref=comm_r.at[s],
            send_sem=rs_send_r.at[s], recv_sem=rs_recv_r.at[s],
            device_id=(right,), device_id_type=pl.DeviceIdType.MESH,
        )
        rdma_l = pltpu.make_async_remote_copy(
            src_ref=src_l, dst_ref=comm_l.at[s],
            send_sem=rs_send_l.at[s], recv_sem=rs_recv_l.at[s],
            device_id=(left,), device_id_type=pl.DeviceIdType.MESH,
        )
        rdma_r.start()
        rdma_l.start()
        rdma_r.wait()
        rdma_l.wait()

    own_r = lax.rem(d + 1, N_DEV)
    own_l = lax.rem(d - 1 + N_DEV, N_DEV)
    tot_r = comm_r[N_DEV - 2].astype(F32) + p_ref[rows(own_r), ca_r].astype(F32)
    tot_l = comm_l[N_DEV - 2].astype(F32) + p_ref[rows(own_l), ca_l].astype(F32)
    g = g_ref[...]
    out_ref[rows(own_r), ca_r] = (
        res_ref[rows(own_r), ca_r].astype(F32) + g[:, :HALF] * tot_r
    ).astype(BF)
    out_ref[rows(own_l), ca_l] = (
        res_ref[rows(own_l), ca_l].astype(F32) + g[:, HALF:] * tot_l
    ).astype(BF)

    for t in range(N_DEV - 1):
        cr = lax.rem(d + 1 - t + N_DEV, N_DEV)
        cl = lax.rem(d - 1 + t + N_DEV, N_DEV)
        rdma_r = pltpu.make_async_remote_copy(
            src_ref=out_ref.at[rows(cr), ca_r],
            dst_ref=out_ref.at[rows(cr), ca_r],
            send_sem=ag_send_r.at[t], recv_sem=ag_recv_r.at[t],
            device_id=(right,), device_id_type=pl.DeviceIdType.MESH,
        )
        rdma_l = pltpu.make_async_remote_copy(
            src_ref=out_ref.at[rows(cl), ca_l],
            dst_ref=out_ref.at[rows(cl), ca_l],
            send_sem=ag_send_l.at[t], recv_sem=ag_recv_l.at[t],
            device_id=(left,), device_id_type=pl.DeviceIdType.MESH,
        )
        rdma_r.start()
        rdma_l.start()
        rdma_r.wait()
        rdma_l.wait()

    def _second(second_barrier):
        for nbr in (left, right):
            pl.semaphore_signal(second_barrier, inc=1, device_id=(nbr,),
                                device_id_type=pl.DeviceIdType.MESH)
        pl.semaphore_wait(second_barrier, 2)

    pl.run_scoped(_second, second_barrier=pltpu.SemaphoreType.REGULAR)


def _all_reduce(partial, res, gate, collective_id):
    return pl.pallas_call(
        _ar_body,
        out_shape=jax.ShapeDtypeStruct((S, D), BF),
        in_specs=[pl.BlockSpec(memory_space=pltpu.VMEM)] * 3,
        out_specs=pl.BlockSpec(memory_space=pltpu.VMEM),
        scratch_shapes=[
            pltpu.VMEM((N_DEV - 1, BLK, HALF), BF),
            pltpu.VMEM((N_DEV - 1, BLK, HALF), BF),
        ] + [pltpu.SemaphoreType.DMA((N_DEV - 1,))] * 8,
        compiler_params=pltpu.CompilerParams(collective_id=collective_id),
    )(partial, res, gate)


def kernel(x, Wq, Wk, Wv, Wo, t_emb, W_mod, W_ff1, W_ff2):
    x0 = x[0]
    mod = t_emb @ W_mod
    sa, sha, ga, sm, shm, gm = jnp.split(mod, 6, axis=-1)

    Wq_b = Wq.astype(BF)
    Wk_b = Wk.astype(BF)
    Wv_b = Wv.astype(BF)
    Wo_b = Wo.astype(BF)
    W1_b = W_ff1.astype(BF)
    W2_b = W_ff2.astype(BF)

    row = pl.BlockSpec((BLK, D), lambda i: (i, 0))
    vec = pl.BlockSpec((1, D), lambda i: (0, 0))
    wfull = pl.BlockSpec((D, D), lambda i: (0, 0))

    q, k, v = pl.pallas_call(
        _qkv_body,
        grid=(NB,),
        in_specs=[row, vec, vec, wfull, wfull, wfull],
        out_specs=[row, row, row],
        out_shape=[jax.ShapeDtypeStruct((S, D), BF)] * 3,
    )(x0, sa, sha, Wq_b, Wk_b, Wv_b)

    attn = pl.pallas_call(
        _attn_body,
        grid=(H_LOC, NB),
        in_specs=[
            pl.BlockSpec((BLK, DH), lambda h, i: (i, h)),
            pl.BlockSpec((S, DH), lambda h, i: (0, h)),
            pl.BlockSpec((S, DH), lambda h, i: (0, h)),
        ],
        out_specs=pl.BlockSpec((BLK, DH), lambda h, i: (i, h)),
        out_shape=jax.ShapeDtypeStruct((S, D), BF),
    )(q, k, v)

    partial1 = pl.pallas_call(
        _proj_body,
        grid=(NB,),
        in_specs=[row, wfull],
        out_specs=row,
        out_shape=jax.ShapeDtypeStruct((S, D), BF),
    )(attn, Wo_b)

    x1 = _all_reduce(partial1, x0, ga, collective_id=0)

    partial2 = pl.pallas_call(
        _ffn_body,
        grid=(NB,),
        in_specs=[
            row, vec, vec,
            pl.BlockSpec((D, 4 * D // N_DEV), lambda i: (0, 0)),
            pl.BlockSpec((4 * D // N_DEV, D), lambda i: (0, 0)),
        ],
        out_specs=row,
        out_shape=jax.ShapeDtypeStruct((S, D), BF),
    )(x1, sm, shm, W1_b, W2_b)

    out = _all_reduce(partial2, x1, gm, collective_id=1)
    return out.astype(F32)[None]
